# baseline (speedup 1.0000x reference)
import functools
import math

import jax
import jax.numpy as jnp
from jax import lax
from jax.experimental import pallas as pl
from jax.experimental.pallas import tpu as pltpu
from jax.experimental.pallas import tpu_sc as plsc

VOCAB_SIZE = 1000000
D = 64
SCALE = math.sqrt(D)

NC = 2
NS = 16
NW = NC * NS

CHUNK = 128


def _embed_body(idx_hbm, table_hbm, out_hbm,
                idx_v, g0, g1, s0, s1,
                semg0, semg1, sems0, sems1,
                *, b_per_w, n_chunks):
    gbufs = (g0, g1)
    sbufs = (s0, s1)
    gsems = (semg0, semg1)
    ssems = (sems0, sems1)

    wid = lax.axis_index("s") * NC + lax.axis_index("c")
    base = wid * b_per_w

    pltpu.sync_copy(idx_hbm.at[pl.ds(base, b_per_w)], idx_v)

    def start_gather(c, b):
        pltpu.async_copy(
            table_hbm.at[idx_v.at[pl.ds(c * CHUNK, CHUNK)]], gbufs[b], gsems[b])

    def wait_gather(b):
        pltpu.make_async_copy(
            table_hbm.at[pl.ds(0, CHUNK)], gbufs[b], gsems[b]).wait()

    def start_store(c, b):
        pltpu.async_copy(
            sbufs[b], out_hbm.at[pl.ds(base + c * CHUNK, CHUNK)], ssems[b])

    def wait_store(b):
        pltpu.make_async_copy(
            sbufs[b], out_hbm.at[pl.ds(0, CHUNK)], ssems[b]).wait()

    start_gather(0, 0)
    start_gather(1, 1)

    @pl.loop(0, n_chunks, step=2)
    def _(g):
        for b in range(2):
            c = g + b
            wait_gather(b)

            @pl.when(c >= 2)
            def _():
                wait_store(b)

            src = gbufs[b]
            dst = sbufs[b]

            @pl.loop(0, CHUNK)
            def _(r):
                for q in range(D // 16):
                    sl = pl.ds(q * 16, 16)
                    dst[r, sl] = src[r, sl] * SCALE

            @pl.when(c + 2 < n_chunks)
            def _():
                start_gather(c + 2, b)

            start_store(c, b)

    wait_store(0)
    wait_store(1)


def kernel(x, table):
    orig_shape = x.shape
    idx = x.reshape(-1).astype(jnp.int32)
    B = idx.shape[0]
    assert B % (NW * CHUNK) == 0
    b_per_w = B // NW
    n_chunks = b_per_w // CHUNK

    mesh = plsc.VectorSubcoreMesh(core_axis_name="c", subcore_axis_name="s")
    out = pl.kernel(
        functools.partial(_embed_body, b_per_w=b_per_w, n_chunks=n_chunks),
        out_type=jax.ShapeDtypeStruct((B, D), jnp.float32),
        mesh=mesh,
        compiler_params=pltpu.CompilerParams(use_tc_tiling_on_sc=False),
        scratch_types=[
            pltpu.VMEM((b_per_w,), jnp.int32),
            pltpu.VMEM((CHUNK, D), jnp.float32),
            pltpu.VMEM((CHUNK, D), jnp.float32),
            pltpu.VMEM((CHUNK, D), jnp.float32),
            pltpu.VMEM((CHUNK, D), jnp.float32),
            pltpu.SemaphoreType.DMA,
            pltpu.SemaphoreType.DMA,
            pltpu.SemaphoreType.DMA,
            pltpu.SemaphoreType.DMA,
        ],
    )(idx, table)
    return out.reshape(*orig_shape, D)


# speedup vs baseline: 1.0221x; 1.0221x over previous
import functools
import math

import jax
import jax.numpy as jnp
from jax import lax
from jax.experimental import pallas as pl
from jax.experimental.pallas import tpu as pltpu
from jax.experimental.pallas import tpu_sc as plsc

VOCAB_SIZE = 1000000
D = 64
SCALE = math.sqrt(D)

NC = 2
NS = 16
NW = NC * NS

CHUNK = 256
NBUF = 4


def _embed_body(idx_hbm, table_hbm, out_hbm,
                idx_v, b0, b1, b2, b3,
                semg0, semg1, semg2, semg3,
                sems0, sems1, sems2, sems3,
                *, b_per_w, n_chunks):
    bufs = (b0, b1, b2, b3)
    gsems = (semg0, semg1, semg2, semg3)
    ssems = (sems0, sems1, sems2, sems3)

    wid = lax.axis_index("s") * NC + lax.axis_index("c")
    base = wid * b_per_w

    pltpu.sync_copy(idx_hbm.at[pl.ds(base, b_per_w)], idx_v)

    def start_gather(c, b):
        pltpu.async_copy(
            table_hbm.at[idx_v.at[pl.ds(c * CHUNK, CHUNK)]], bufs[b], gsems[b])

    def wait_gather(b):
        pltpu.make_async_copy(
            table_hbm.at[pl.ds(0, CHUNK)], bufs[b], gsems[b]).wait()

    def start_store(c, b):
        pltpu.async_copy(
            bufs[b], out_hbm.at[pl.ds(base + c * CHUNK, CHUNK)], ssems[b])

    def wait_store(b):
        pltpu.make_async_copy(
            bufs[b], out_hbm.at[pl.ds(0, CHUNK)], ssems[b]).wait()

    start_gather(0, 0)
    start_gather(1, 1)

    @pl.loop(0, n_chunks, step=NBUF)
    def _(g):
        for b in range(NBUF):
            c = g + b
            wait_gather(b)

            buf = bufs[b]

            @plsc.parallel_loop(0, CHUNK, unroll=8)
            def _(r):
                for q in range(D // 16):
                    sl = pl.ds(q * 16, 16)
                    buf[r, sl] = buf[r, sl] * SCALE

            start_store(c, b)

            b2 = (b + 2) % NBUF
            c2 = c + 2

            @pl.when(c2 < n_chunks)
            def _():
                @pl.when(c2 >= NBUF)
                def _():
                    wait_store(b2)

                start_gather(c2, b2)

    for b in range(NBUF):
        wait_store(b)


def kernel(x, table):
    orig_shape = x.shape
    idx = x.reshape(-1).astype(jnp.int32)
    B = idx.shape[0]
    assert B % (NW * CHUNK * NBUF) == 0
    b_per_w = B // NW
    n_chunks = b_per_w // CHUNK

    mesh = plsc.VectorSubcoreMesh(core_axis_name="c", subcore_axis_name="s")
    out = pl.kernel(
        functools.partial(_embed_body, b_per_w=b_per_w, n_chunks=n_chunks),
        out_type=jax.ShapeDtypeStruct((B, D), jnp.float32),
        mesh=mesh,
        compiler_params=pltpu.CompilerParams(use_tc_tiling_on_sc=False),
        scratch_types=(
            [pltpu.VMEM((b_per_w,), jnp.int32)]
            + [pltpu.VMEM((CHUNK, D), jnp.float32) for _ in range(NBUF)]
            + [pltpu.SemaphoreType.DMA for _ in range(2 * NBUF)]
        ),
    )(idx, table)
    return out.reshape(*orig_shape, D)
